# Initial kernel scaffold; baseline (speedup 1.0000x reference)
import jax
import jax.numpy as jnp
from jax.experimental import pallas as pl


def kernel(x, w1, b1, w2, b2, s1e, s1o, s2e, s2o, wf1, bf1, wf2, bf2, wf3, bf3):
    raise NotImplementedError("write your pallas kernel here")


# speedup vs baseline: 18.5299x; 18.5299x over previous
import jax
import jax.numpy as jnp
from jax.experimental import pallas as pl
from jax.experimental.pallas import tpu as pltpu


def _lenet_body(x_ref, w1_ref, b1_ref, w2_ref, b2_ref,
                wf1_ref, bf1_ref, wf2_ref, bf2_ref, wf3_ref, bf3_ref,
                o_ref, xsc_ref, acc1_ref, xs2_ref, acc2_ref):
    B = o_ref.shape[1]
    f32 = jnp.float32

    for ci in range(3):
        for j in range(5):
            xsc_ref[ci, j] = x_ref[ci, j:j + 1024, :]
    acc1_ref[...] = jnp.zeros((6, 896, B), f32)

    def c1_body(t, carry):
        ii = t // 5
        jj = t % 5
        base = pl.multiple_of(ii * 32, 32)
        srcs = [xsc_ref[ci, jj, pl.ds(base, 896), :] for ci in range(3)]
        for co in range(6):
            s = (w1_ref[t, 0, co] * srcs[0]
                 + w1_ref[t, 1, co] * srcs[1]
                 + w1_ref[t, 2, co] * srcs[2])
            acc1_ref[co] = acc1_ref[co] + s
        return carry

    jax.lax.fori_loop(0, 25, c1_body, 0)

    zpad = jnp.zeros((8, B), f32)
    for co in range(6):
        y = jnp.maximum(acc1_ref[co] + b1_ref[0, co], 0.0)
        y = y.reshape(14, 2, 32, B)
        v = jnp.maximum(y[:, 0], y[:, 1])
        v = v.reshape(14, 16, 2, B)
        p = jnp.maximum(v[:, :, 0], v[:, :, 1])
        p = jnp.concatenate([p.reshape(224, B), zpad], 0)
        for j in range(5):
            xs2_ref[co, j] = p[j:j + 224, :]

    acc2_ref[...] = jnp.zeros((16, 160, B), f32)

    def c2_body(t, carry):
        ii = t // 5
        jj = t % 5
        base = pl.multiple_of(ii * 16, 16)
        srcs = [xs2_ref[ci, jj, pl.ds(base, 160), :] for ci in range(6)]
        for co in range(16):
            s = w2_ref[t, 0, co] * srcs[0]
            for ci in range(1, 6):
                s = s + w2_ref[t, ci, co] * srcs[ci]
            acc2_ref[co] = acc2_ref[co] + s
        return carry

    jax.lax.fori_loop(0, 25, c2_body, 0)

    feats = []
    for co in range(16):
        y = jnp.maximum(acc2_ref[co] + b2_ref[0, co], 0.0)
        y = y.reshape(5, 2, 16, B)
        v = jnp.maximum(y[:, 0], y[:, 1])
        v = v.reshape(5, 8, 2, B)
        p = jnp.maximum(v[:, :, 0], v[:, :, 1])
        feats.append(p.reshape(40, B))

    feat = jnp.concatenate(feats, axis=0)

    h = jnp.dot(wf1_ref[...], feat, preferred_element_type=f32) + bf1_ref[...]
    h = jnp.maximum(h, 0.0)
    h = jnp.dot(wf2_ref[...], h, preferred_element_type=f32) + bf2_ref[...]
    h = jnp.maximum(h, 0.0)
    o = jnp.dot(wf3_ref[...], h, preferred_element_type=f32) + bf3_ref[...]
    o_ref[...] = o


def kernel(x, w1, b1, w2, b2, s1e, s1o, s2e, s2o,
           wf1, bf1, wf2, bf2, wf3, bf3):
    n = x.shape[0]
    B = 128
    npad = -(-n // B) * B

    xt = jnp.transpose(x, (1, 2, 3, 0)).reshape(3, 1024, n)
    xt = jnp.pad(xt, ((0, 0), (0, 8), (0, npad - n)))

    wf1t = wf1.T
    wf2t = wf2.T
    wf3t = wf3.T
    bf1t = jnp.broadcast_to(bf1.reshape(128, 1), (128, B)) * jnp.float32(1)
    bf2t = jnp.broadcast_to(bf2.reshape(128, 1), (128, B)) * jnp.float32(1)
    bf3t = jnp.broadcast_to(bf3.reshape(128, 1), (128, B)) * jnp.float32(1)

    smem = pl.BlockSpec(memory_space=pltpu.SMEM)
    out = pl.pallas_call(
        _lenet_body,
        out_shape=jax.ShapeDtypeStruct((128, npad), jnp.float32),
        grid=(npad // B,),
        in_specs=[
            pl.BlockSpec((3, 1032, B), lambda b: (0, 0, b)),
            smem,
            smem,
            smem,
            smem,
            pl.BlockSpec((128, 640), lambda b: (0, 0)),
            pl.BlockSpec((128, B), lambda b: (0, 0)),
            pl.BlockSpec((128, 128), lambda b: (0, 0)),
            pl.BlockSpec((128, B), lambda b: (0, 0)),
            pl.BlockSpec((128, 128), lambda b: (0, 0)),
            pl.BlockSpec((128, B), lambda b: (0, 0)),
        ],
        out_specs=pl.BlockSpec((128, B), lambda b: (0, b)),
        scratch_shapes=[
            pltpu.VMEM((3, 5, 1024, B), jnp.float32),
            pltpu.VMEM((6, 896, B), jnp.float32),
            pltpu.VMEM((6, 5, 224, B), jnp.float32),
            pltpu.VMEM((16, 160, B), jnp.float32),
        ],
        compiler_params=pltpu.CompilerParams(
            dimension_semantics=("parallel",)),
    )(xt, w1, b1, w2, b2, wf1t, bf1t, wf2t, bf2t, wf3t, bf3t)

    return jax.lax.slice(out, (0, 0), (10, n)).T
